# baseline (speedup 1.0000x reference)
import functools

import jax
import jax.numpy as jnp
from jax import lax
from jax.experimental import pallas as pl
from jax.experimental.pallas import tpu as pltpu
from jax.experimental.pallas import tpu_sc as plsc

_N = 100000
_E = 1600000
_H = 32
_G = 64

_RB = 2000
_MB = 4000

_K = 400
_CH = (_E // 16) // _K
_NBUF = 4
_TPS = _N // 16


def _zeros16f():
    return jnp.zeros((16,), jnp.float32)


def _splat16(v, t):
    idx = (jnp.zeros((16,), jnp.int32) + t)[:, None]
    return lax.gather(
        v, idx,
        lax.GatherDimensionNumbers(offset_dims=(), collapsed_slice_dims=(0,),
                                   start_index_map=(0,)),
        (1,), mode=lax.GatherScatterMode.PROMISE_IN_BOUNDS)


def _sc_edge_agg(src, dst, ew, g2=None):
    with_g = g2 is not None
    mesh = plsc.VectorSubcoreMesh(core_axis_name="c", subcore_axis_name="s")

    nb = _NBUF
    scratch = (
        [pltpu.VMEM_SHARED((_N, 16), jnp.float32)]
        + [pltpu.VMEM((_K, 16), jnp.float32) for _ in range(nb)]
        + [pltpu.VMEM((_K,), jnp.int32) for _ in range(nb)]
        + [pltpu.VMEM((_K,), jnp.int32) for _ in range(nb)]
        + [pltpu.VMEM((_K,), jnp.float32) for _ in range(nb)]
        + [pltpu.SemaphoreType.DMA for _ in range(3 * nb)]
    )

    @functools.partial(
        pl.kernel,
        out_type=jax.ShapeDtypeStruct((2 * _N, 16), jnp.float32),
        mesh=mesh,
        compiler_params=pltpu.CompilerParams(use_tc_tiling_on_sc=False),
        scratch_types=scratch,
    )
    def k(*refs):
        if with_g:
            src_hbm, dst_hbm, ew_hbm, g_hbm, out_hbm, acc = refs[:6]
            rest = refs[6:]
        else:
            src_hbm, dst_hbm, ew_hbm, out_hbm, acc = refs[:5]
            rest = refs[5:]
        rows = rest[0 * nb:1 * nb]
        srcv = rest[1 * nb:2 * nb]
        dstv = rest[2 * nb:3 * nb]
        ewv = rest[3 * nb:4 * nb]
        base_s = 4 * nb
        isem = rest[base_s + 0 * nb:base_s + 1 * nb]
        gsem = rest[base_s + 1 * nb:base_s + 2 * nb]
        ssem = rest[base_s + 2 * nb:base_s + 3 * nb]
        c = lax.axis_index("c")
        s = lax.axis_index("s")
        ebase = s * (_K * _CH)

        zrow = rows[0]

        @plsc.parallel_loop(0, _K, 1, unroll=8)
        def _(r):
            zrow[r, pl.ds(0, 16)] = _zeros16f()

        nfull, tail = divmod(_TPS, _K)
        for q in range(nfull):
            pltpu.sync_copy(zrow, acc.at[pl.ds(s * _TPS + q * _K, _K)])
        if tail:
            pltpu.sync_copy(zrow.at[pl.ds(0, tail)],
                            acc.at[pl.ds(s * _TPS + nfull * _K, tail)])
        plsc.subcore_barrier()

        def idx_copies(cidx, b):
            base = ebase + cidx * _K
            pltpu.async_copy(src_hbm.at[pl.ds(base, _K)], srcv[b], isem[b])
            pltpu.async_copy(dst_hbm.at[pl.ds(base, _K)], dstv[b], isem[b])
            pltpu.async_copy(ew_hbm.at[pl.ds(base, _K)], ewv[b], isem[b])

        def wait_scatter(b):
            pltpu.make_async_copy(rows[b], acc.at[dstv[b]], ssem[b]).wait()

        def issue_gather(cidx, b):
            base = ebase + cidx * _K
            pltpu.make_async_copy(src_hbm.at[pl.ds(base, _K)], srcv[b],
                                  isem[b]).wait()
            pltpu.make_async_copy(dst_hbm.at[pl.ds(base, _K)], dstv[b],
                                  isem[b]).wait()
            pltpu.make_async_copy(ew_hbm.at[pl.ds(base, _K)], ewv[b],
                                  isem[b]).wait()
            if with_g:
                @plsc.parallel_loop(0, _K, 16, unroll=2)
                def _(m):
                    srcv[b][pl.ds(m, 16)] = srcv[b][pl.ds(m, 16)] + c * _N

                pltpu.async_copy(g_hbm.at[srcv[b]], rows[b], gsem[b])

        def process(b):
            if with_g:
                pltpu.make_async_copy(g_hbm.at[srcv[b]], rows[b],
                                      gsem[b]).wait()

            @plsc.parallel_loop(0, _K, 16, unroll=2)
            def _(m):
                w16 = ewv[b][pl.ds(m, 16)]
                for t in range(16):
                    e = m + t
                    wk = _splat16(w16, t)
                    if with_g:
                        rows[b][e, pl.ds(0, 16)] = (
                            rows[b][e, pl.ds(0, 16)] * wk)
                    else:
                        rows[b][e, pl.ds(0, 16)] = (
                            rows[b][e, pl.ds(0, 16)] * 0.0 + wk)

            pltpu.async_copy(rows[b], acc.at[dstv[b]], ssem[b], add=True)

        idx_copies(jnp.int32(0), 0)
        idx_copies(jnp.int32(1), 1)
        idx_copies(jnp.int32(2), 2)
        issue_gather(jnp.int32(0), 0)
        issue_gather(jnp.int32(1), 1)

        niter = _CH // nb

        @pl.loop(0, niter)
        def _(i):
            for b in range(nb):
                cidx = i * nb + b
                process(b)
                g_b = (b + 2) % nb
                i_b = (b + 3) % nb
                issue_gather(cidx + 2, g_b)
                if b < 2:
                    @pl.when(i > 0)
                    def _():
                        wait_scatter(i_b)
                    idx_copies(cidx + 3, i_b)
                elif b == 2:
                    wait_scatter(i_b)
                    idx_copies(cidx + 3, i_b)
                else:
                    @pl.when(i < niter - 1)
                    def _():
                        wait_scatter(i_b)
                        idx_copies(cidx + 3, i_b)

        process(0)
        process(1)
        for b in range(nb):
            wait_scatter(b)
        wait_scatter(0)
        plsc.subcore_barrier()

        pltpu.sync_copy(acc.at[pl.ds(s * _TPS, _TPS)],
                        out_hbm.at[pl.ds(c * _N + s * _TPS, _TPS)])

    if with_g:
        out = k(src, dst, ew, g2.reshape(2 * _N, 16))
    else:
        out = k(src, dst, ew)
    return out.reshape(2, _N, 16)


def _dinv_from_deg(deg2):
    def body(p_ref, o_ref):
        o_ref[...] = lax.rsqrt(p_ref[0][:, :1] + 1.0)

    return pl.pallas_call(
        body,
        grid=(_N // _MB,),
        in_specs=[pl.BlockSpec((1, _MB, 16), lambda i: (0, i, 0))],
        out_specs=pl.BlockSpec((_MB, 1), lambda i: (i, 0)),
        out_shape=jax.ShapeDtypeStruct((_N, 1), jnp.float32),
    )(deg2)


def _mm1(x, W, dinv_col):
    def body(x_ref, w_ref, d_ref, o_ref):
        h = jnp.dot(x_ref[...], w_ref[...],
                    preferred_element_type=jnp.float32) * d_ref[...]
        o_ref[0] = h[:, :16]
        o_ref[1] = h[:, 16:]

    return pl.pallas_call(
        body,
        grid=(_N // _RB,),
        in_specs=[
            pl.BlockSpec((_RB, x.shape[1]), lambda i: (i, 0)),
            pl.BlockSpec((x.shape[1], _H), lambda i: (0, 0)),
            pl.BlockSpec((_RB, 1), lambda i: (i, 0)),
        ],
        out_specs=pl.BlockSpec((2, _RB, 16), lambda i: (0, i, 0)),
        out_shape=jax.ShapeDtypeStruct((2, _N, 16), jnp.float32),
    )(x, W, dinv_col)


def _mm_fused(agg2, g2, dinv_col, bias, W):
    def body(a_ref, g_ref, d_ref, b_ref, w_ref, o_ref):
        a = jnp.concatenate([a_ref[0], a_ref[1]], axis=1)
        g = jnp.concatenate([g_ref[0], g_ref[1]], axis=1)
        d = d_ref[...]
        t = jnp.maximum(d * (a + g) + b_ref[...], 0.0)
        h = jnp.dot(t, w_ref[...], preferred_element_type=jnp.float32) * d
        o_ref[0] = h[:, :16]
        o_ref[1] = h[:, 16:]

    return pl.pallas_call(
        body,
        grid=(_N // _RB,),
        in_specs=[
            pl.BlockSpec((2, _RB, 16), lambda i: (0, i, 0)),
            pl.BlockSpec((2, _RB, 16), lambda i: (0, i, 0)),
            pl.BlockSpec((_RB, 1), lambda i: (i, 0)),
            pl.BlockSpec((1, _H), lambda i: (0, 0)),
            pl.BlockSpec((_H, _H), lambda i: (0, 0)),
        ],
        out_specs=pl.BlockSpec((2, _RB, 16), lambda i: (0, i, 0)),
        out_shape=jax.ShapeDtypeStruct((2, _N, 16), jnp.float32),
    )(agg2, g2, dinv_col, bias, W)


def _pool(agg2, g2, dinv_col, bias, batch_col):
    def body(a_ref, g_ref, d_ref, b_ref, bat_ref, s_ref, c_ref):
        i = pl.program_id(0)
        a = jnp.concatenate([a_ref[0], a_ref[1]], axis=1)
        g = jnp.concatenate([g_ref[0], g_ref[1]], axis=1)
        y = d_ref[...] * (a + g) + b_ref[...]
        onehot = (bat_ref[...] ==
                  lax.broadcasted_iota(jnp.int32, (_RB, _G), 1)
                  ).astype(jnp.float32)
        sums = lax.dot_general(onehot, y, (((0,), (0,)), ((), ())),
                               preferred_element_type=jnp.float32)
        counts = jnp.sum(onehot, axis=0, keepdims=True)

        @pl.when(i == 0)
        def _():
            s_ref[...] = jnp.zeros_like(s_ref)
            c_ref[...] = jnp.zeros_like(c_ref)

        s_ref[...] += sums
        c_ref[...] += counts

    return pl.pallas_call(
        body,
        grid=(_N // _RB,),
        in_specs=[
            pl.BlockSpec((2, _RB, 16), lambda i: (0, i, 0)),
            pl.BlockSpec((2, _RB, 16), lambda i: (0, i, 0)),
            pl.BlockSpec((_RB, 1), lambda i: (i, 0)),
            pl.BlockSpec((1, _H), lambda i: (0, 0)),
            pl.BlockSpec((_RB, 1), lambda i: (i, 0)),
        ],
        out_specs=[
            pl.BlockSpec((_G, _H), lambda i: (0, 0)),
            pl.BlockSpec((1, _G), lambda i: (0, 0)),
        ],
        out_shape=[
            jax.ShapeDtypeStruct((_G, _H), jnp.float32),
            jax.ShapeDtypeStruct((1, _G), jnp.float32),
        ],
    )(agg2, g2, dinv_col, bias, batch_col)


def _mlp_head(sums, counts, M1W, M1b, M2W, M2b, M3W, M3b):
    def body(s_ref, c_ref, w1, b1, w2, b2, w3, b3, o_ref):
        pooled = s_ref[...] / jnp.maximum(c_ref[...], 1.0).reshape(_G, 1)
        z = jnp.maximum(
            jnp.dot(pooled, w1[...], preferred_element_type=jnp.float32)
            + b1[...], 0.0)
        z = jnp.maximum(
            jnp.dot(z, w2[...], preferred_element_type=jnp.float32)
            + b2[...], 0.0)
        o_ref[...] = (
            jnp.dot(z, w3[...], preferred_element_type=jnp.float32)
            + b3[...])

    return pl.pallas_call(
        body,
        out_shape=jax.ShapeDtypeStruct((_G, 16), jnp.float32),
    )(sums, counts, M1W, M1b.reshape(1, -1), M2W, M2b.reshape(1, -1),
      M3W, M3b.reshape(1, -1))


def kernel(x, edge_index, edge_weight, batch,
           W1, b1, W2, b2, W3, b3, M1W, M1b, M2W, M2b, M3W, M3b):
    src, dst = edge_index[0], edge_index[1]
    deg2 = _sc_edge_agg(src, dst, edge_weight)
    dinv_col = _dinv_from_deg(deg2)

    g2 = _mm1(x, W1, dinv_col)
    agg2 = _sc_edge_agg(src, dst, edge_weight, g2)
    g2b = _mm_fused(agg2, g2, dinv_col, b1.reshape(1, -1), W2)
    agg2b = _sc_edge_agg(src, dst, edge_weight, g2b)
    g2c = _mm_fused(agg2b, g2b, dinv_col, b2.reshape(1, -1), W3)
    agg2c = _sc_edge_agg(src, dst, edge_weight, g2c)

    sums, counts = _pool(agg2c, g2c, dinv_col, b3.reshape(1, -1),
                         batch[:, None])
    return _mlp_head(sums, counts, M1W, M1b, M2W, M2b, M3W, M3b)


# speedup vs baseline: 1.0034x; 1.0034x over previous
import functools

import jax
import jax.numpy as jnp
from jax import lax
from jax.experimental import pallas as pl
from jax.experimental.pallas import tpu as pltpu
from jax.experimental.pallas import tpu_sc as plsc

_N = 100000
_E = 1600000
_H = 32
_G = 64

_RB = 2000
_MB = 4000

_K = 400
_CH = (_E // 16) // _K
_NBUF = 4
_TPS = _N // 16


def _zeros16f():
    return jnp.zeros((16,), jnp.float32)


def _splat16(v, t):
    idx = (jnp.zeros((16,), jnp.int32) + t)[:, None]
    return lax.gather(
        v, idx,
        lax.GatherDimensionNumbers(offset_dims=(), collapsed_slice_dims=(0,),
                                   start_index_map=(0,)),
        (1,), mode=lax.GatherScatterMode.PROMISE_IN_BOUNDS)


def _sc_edge_agg(src, dst, ew, g2=None):
    with_g = g2 is not None
    mesh = plsc.VectorSubcoreMesh(core_axis_name="c", subcore_axis_name="s")

    nb = _NBUF
    scratch = (
        [pltpu.VMEM_SHARED((_N, 16), jnp.float32)]
        + [pltpu.VMEM((_K, 16), jnp.float32) for _ in range(nb)]
        + [pltpu.VMEM((_K,), jnp.int32) for _ in range(nb)]
        + [pltpu.VMEM((_K,), jnp.int32) for _ in range(nb)]
        + [pltpu.VMEM((_K,), jnp.float32) for _ in range(nb)]
        + [pltpu.SemaphoreType.DMA for _ in range(3 * nb)]
    )

    @functools.partial(
        pl.kernel,
        out_type=jax.ShapeDtypeStruct((2 * _N, 16), jnp.float32),
        mesh=mesh,
        compiler_params=pltpu.CompilerParams(use_tc_tiling_on_sc=False),
        scratch_types=scratch,
    )
    def k(*refs):
        if with_g:
            src_hbm, dst_hbm, ew_hbm, g_hbm, out_hbm, acc = refs[:6]
            rest = refs[6:]
        else:
            src_hbm, dst_hbm, ew_hbm, out_hbm, acc = refs[:5]
            rest = refs[5:]
        rows = rest[0 * nb:1 * nb]
        srcv = rest[1 * nb:2 * nb]
        dstv = rest[2 * nb:3 * nb]
        ewv = rest[3 * nb:4 * nb]
        base_s = 4 * nb
        isem = rest[base_s + 0 * nb:base_s + 1 * nb]
        gsem = rest[base_s + 1 * nb:base_s + 2 * nb]
        ssem = rest[base_s + 2 * nb:base_s + 3 * nb]
        c = lax.axis_index("c")
        s = lax.axis_index("s")
        ebase = s * (_K * _CH)

        zrow = rows[0]

        @plsc.parallel_loop(0, _K, 1, unroll=8)
        def _(r):
            zrow[r, pl.ds(0, 16)] = _zeros16f()

        nfull, tail = divmod(_TPS, _K)
        for q in range(nfull):
            pltpu.sync_copy(zrow, acc.at[pl.ds(s * _TPS + q * _K, _K)])
        if tail:
            pltpu.sync_copy(zrow.at[pl.ds(0, tail)],
                            acc.at[pl.ds(s * _TPS + nfull * _K, tail)])
        plsc.subcore_barrier()

        def idx_copies(cidx, b):
            base = ebase + cidx * _K
            pltpu.async_copy(src_hbm.at[pl.ds(base, _K)], srcv[b], isem[b])
            pltpu.async_copy(dst_hbm.at[pl.ds(base, _K)], dstv[b], isem[b])
            pltpu.async_copy(ew_hbm.at[pl.ds(base, _K)], ewv[b], isem[b])

        def wait_scatter(b):
            pltpu.make_async_copy(rows[b], acc.at[dstv[b]], ssem[b]).wait()

        def issue_gather(cidx, b):
            base = ebase + cidx * _K
            pltpu.make_async_copy(src_hbm.at[pl.ds(base, _K)], srcv[b],
                                  isem[b]).wait()
            pltpu.make_async_copy(dst_hbm.at[pl.ds(base, _K)], dstv[b],
                                  isem[b]).wait()
            pltpu.make_async_copy(ew_hbm.at[pl.ds(base, _K)], ewv[b],
                                  isem[b]).wait()
            if with_g:
                @plsc.parallel_loop(0, _K, 16, unroll=2)
                def _(m):
                    srcv[b][pl.ds(m, 16)] = srcv[b][pl.ds(m, 16)] + c * _N

                pltpu.async_copy(g_hbm.at[srcv[b]], rows[b], gsem[b])

        def process(b):
            if with_g:
                pltpu.make_async_copy(g_hbm.at[srcv[b]], rows[b],
                                      gsem[b]).wait()

            @plsc.parallel_loop(0, _K, 16)
            def _(m):
                w16 = ewv[b][pl.ds(m, 16)]
                for t in range(16):
                    e = m + t
                    wk = _splat16(w16, t)
                    if with_g:
                        rows[b][e, pl.ds(0, 16)] = (
                            rows[b][e, pl.ds(0, 16)] * wk)
                    else:
                        rows[b][e, pl.ds(0, 16)] = (
                            rows[b][e, pl.ds(0, 16)] * 0.0 + wk)

            pltpu.async_copy(rows[b], acc.at[dstv[b]], ssem[b], add=True)

        idx_copies(jnp.int32(0), 0)
        idx_copies(jnp.int32(1), 1)
        idx_copies(jnp.int32(2), 2)
        issue_gather(jnp.int32(0), 0)
        issue_gather(jnp.int32(1), 1)

        niter = _CH // nb

        @pl.loop(0, niter)
        def _(i):
            for b in range(nb):
                cidx = i * nb + b
                process(b)
                g_b = (b + 2) % nb
                i_b = (b + 3) % nb
                issue_gather(cidx + 2, g_b)
                if b < 2:
                    @pl.when(i > 0)
                    def _():
                        wait_scatter(i_b)
                    idx_copies(cidx + 3, i_b)
                elif b == 2:
                    wait_scatter(i_b)
                    idx_copies(cidx + 3, i_b)
                else:
                    @pl.when(i < niter - 1)
                    def _():
                        wait_scatter(i_b)
                        idx_copies(cidx + 3, i_b)

        process(0)
        process(1)
        for b in range(nb):
            wait_scatter(b)
        wait_scatter(0)
        plsc.subcore_barrier()

        pltpu.sync_copy(acc.at[pl.ds(s * _TPS, _TPS)],
                        out_hbm.at[pl.ds(c * _N + s * _TPS, _TPS)])

    if with_g:
        out = k(src, dst, ew, g2.reshape(2 * _N, 16))
    else:
        out = k(src, dst, ew)
    return out.reshape(2, _N, 16)


def _dinv_from_deg(deg2):
    def body(p_ref, o_ref):
        o_ref[...] = lax.rsqrt(p_ref[0][:, :1] + 1.0)

    return pl.pallas_call(
        body,
        grid=(_N // _MB,),
        in_specs=[pl.BlockSpec((1, _MB, 16), lambda i: (0, i, 0))],
        out_specs=pl.BlockSpec((_MB, 1), lambda i: (i, 0)),
        out_shape=jax.ShapeDtypeStruct((_N, 1), jnp.float32),
    )(deg2)


def _mm1(x, W, dinv_col):
    def body(x_ref, w_ref, d_ref, o_ref):
        h = jnp.dot(x_ref[...], w_ref[...],
                    preferred_element_type=jnp.float32) * d_ref[...]
        o_ref[0] = h[:, :16]
        o_ref[1] = h[:, 16:]

    return pl.pallas_call(
        body,
        grid=(_N // _RB,),
        in_specs=[
            pl.BlockSpec((_RB, x.shape[1]), lambda i: (i, 0)),
            pl.BlockSpec((x.shape[1], _H), lambda i: (0, 0)),
            pl.BlockSpec((_RB, 1), lambda i: (i, 0)),
        ],
        out_specs=pl.BlockSpec((2, _RB, 16), lambda i: (0, i, 0)),
        out_shape=jax.ShapeDtypeStruct((2, _N, 16), jnp.float32),
    )(x, W, dinv_col)


def _mm_fused(agg2, g2, dinv_col, bias, W):
    def body(a_ref, g_ref, d_ref, b_ref, w_ref, o_ref):
        a = jnp.concatenate([a_ref[0], a_ref[1]], axis=1)
        g = jnp.concatenate([g_ref[0], g_ref[1]], axis=1)
        d = d_ref[...]
        t = jnp.maximum(d * (a + g) + b_ref[...], 0.0)
        h = jnp.dot(t, w_ref[...], preferred_element_type=jnp.float32) * d
        o_ref[0] = h[:, :16]
        o_ref[1] = h[:, 16:]

    return pl.pallas_call(
        body,
        grid=(_N // _RB,),
        in_specs=[
            pl.BlockSpec((2, _RB, 16), lambda i: (0, i, 0)),
            pl.BlockSpec((2, _RB, 16), lambda i: (0, i, 0)),
            pl.BlockSpec((_RB, 1), lambda i: (i, 0)),
            pl.BlockSpec((1, _H), lambda i: (0, 0)),
            pl.BlockSpec((_H, _H), lambda i: (0, 0)),
        ],
        out_specs=pl.BlockSpec((2, _RB, 16), lambda i: (0, i, 0)),
        out_shape=jax.ShapeDtypeStruct((2, _N, 16), jnp.float32),
    )(agg2, g2, dinv_col, bias, W)


def _pool(agg2, g2, dinv_col, bias, batch_col):
    def body(a_ref, g_ref, d_ref, b_ref, bat_ref, s_ref, c_ref):
        i = pl.program_id(0)
        a = jnp.concatenate([a_ref[0], a_ref[1]], axis=1)
        g = jnp.concatenate([g_ref[0], g_ref[1]], axis=1)
        y = d_ref[...] * (a + g) + b_ref[...]
        onehot = (bat_ref[...] ==
                  lax.broadcasted_iota(jnp.int32, (_RB, _G), 1)
                  ).astype(jnp.float32)
        sums = lax.dot_general(onehot, y, (((0,), (0,)), ((), ())),
                               preferred_element_type=jnp.float32)
        counts = jnp.sum(onehot, axis=0, keepdims=True)

        @pl.when(i == 0)
        def _():
            s_ref[...] = jnp.zeros_like(s_ref)
            c_ref[...] = jnp.zeros_like(c_ref)

        s_ref[...] += sums
        c_ref[...] += counts

    return pl.pallas_call(
        body,
        grid=(_N // _RB,),
        in_specs=[
            pl.BlockSpec((2, _RB, 16), lambda i: (0, i, 0)),
            pl.BlockSpec((2, _RB, 16), lambda i: (0, i, 0)),
            pl.BlockSpec((_RB, 1), lambda i: (i, 0)),
            pl.BlockSpec((1, _H), lambda i: (0, 0)),
            pl.BlockSpec((_RB, 1), lambda i: (i, 0)),
        ],
        out_specs=[
            pl.BlockSpec((_G, _H), lambda i: (0, 0)),
            pl.BlockSpec((1, _G), lambda i: (0, 0)),
        ],
        out_shape=[
            jax.ShapeDtypeStruct((_G, _H), jnp.float32),
            jax.ShapeDtypeStruct((1, _G), jnp.float32),
        ],
    )(agg2, g2, dinv_col, bias, batch_col)


def _mlp_head(sums, counts, M1W, M1b, M2W, M2b, M3W, M3b):
    def body(s_ref, c_ref, w1, b1, w2, b2, w3, b3, o_ref):
        pooled = s_ref[...] / jnp.maximum(c_ref[...], 1.0).reshape(_G, 1)
        z = jnp.maximum(
            jnp.dot(pooled, w1[...], preferred_element_type=jnp.float32)
            + b1[...], 0.0)
        z = jnp.maximum(
            jnp.dot(z, w2[...], preferred_element_type=jnp.float32)
            + b2[...], 0.0)
        o_ref[...] = (
            jnp.dot(z, w3[...], preferred_element_type=jnp.float32)
            + b3[...])

    return pl.pallas_call(
        body,
        out_shape=jax.ShapeDtypeStruct((_G, 16), jnp.float32),
    )(sums, counts, M1W, M1b.reshape(1, -1), M2W, M2b.reshape(1, -1),
      M3W, M3b.reshape(1, -1))


def kernel(x, edge_index, edge_weight, batch,
           W1, b1, W2, b2, W3, b3, M1W, M1b, M2W, M2b, M3W, M3b):
    src, dst = edge_index[0], edge_index[1]
    deg2 = _sc_edge_agg(src, dst, edge_weight)
    dinv_col = _dinv_from_deg(deg2)

    g2 = _mm1(x, W1, dinv_col)
    agg2 = _sc_edge_agg(src, dst, edge_weight, g2)
    g2b = _mm_fused(agg2, g2, dinv_col, b1.reshape(1, -1), W2)
    agg2b = _sc_edge_agg(src, dst, edge_weight, g2b)
    g2c = _mm_fused(agg2b, g2b, dinv_col, b2.reshape(1, -1), W3)
    agg2c = _sc_edge_agg(src, dst, edge_weight, g2c)

    sums, counts = _pool(agg2c, g2c, dinv_col, b3.reshape(1, -1),
                         batch[:, None])
    return _mlp_head(sums, counts, M1W, M1b, M2W, M2b, M3W, M3b)
